# baseline (speedup 1.0000x reference)
import functools

import jax
import jax.numpy as jnp
from jax import lax
from jax.experimental import pallas as pl
from jax.experimental.pallas import tpu as pltpu
from jax.experimental.pallas import tpu_sc as plsc

_BATCH = 4
_SEQ = 8192
_D = 8
_NT = _SEQ // 128
_TBL = _D * _SEQ

_info = plsc.get_sparse_core_info()
_NC = _info.num_cores
_NS = _info.num_subcores
_NACT = 16
_BPW = _BATCH * _SEQ // _NACT
_WPR = _SEQ // _BPW
_BLK = _BPW // 128
_L = 16
_HALF = _BPW * _D // 2

_mesh = plsc.VectorSubcoreMesh(core_axis_name="c", subcore_axis_name="s")


@functools.partial(
    pl.kernel,
    mesh=_mesh,
    out_type=jax.ShapeDtypeStruct((_BATCH, _SEQ * _D), jnp.float32),
    scratch_types=[
        pltpu.VMEM((_BPW,), jnp.int32),
        pltpu.VMEM((_TBL,), jnp.float32),
        pltpu.VMEM((_BPW * _D,), jnp.float32),
        pltpu.SemaphoreType.DMA,
        pltpu.SemaphoreType.DMA,
    ],
    compiler_params=pltpu.CompilerParams(
        use_tc_tiling_on_sc=False, needs_layout_passes=False
    ),
)
def _gather_t(xk_hbm, tk_hbm, out_hbm, idx_v, tbl_v, out_v, sem, wsem):
    wid = lax.axis_index("s") * _NC + lax.axis_index("c")

    @pl.when(wid < _NACT)
    def _():
        b = wid // _WPR
        blk0 = (wid % _WPR) * _BLK
        cps = [pltpu.async_copy(tk_hbm, tbl_v, sem)]
        for i in range(_BLK):
            cps.append(
                pltpu.async_copy(
                    xk_hbm.at[blk0 + i, b], idx_v.at[pl.ds(i * 128, 128)], sem
                )
            )
        for cp in cps:
            cp.wait()

        def body(k2, _):
            for u in range(2):
                k = k2 * 2 + u
                tvec = idx_v[pl.ds(pl.multiple_of(k * _L, _L), _L)]
                base = tvec + lax.shift_right_logical(tvec, 7) * 896
                obase = (k >> 3) * 1024 + (k & 7) * _L
                for d in range(_D):
                    vals = plsc.load_gather(tbl_v, [base + d * 128])
                    out_v[pl.ds(pl.multiple_of(obase + d * 128, _L), _L)] = vals
            return _

        lax.fori_loop(0, _BPW // _L // 4, body, None)
        wb0 = pltpu.async_copy(
            out_v.at[pl.ds(0, _HALF)],
            out_hbm.at[b, pl.ds(blk0 * 1024, _HALF)],
            wsem,
        )
        lax.fori_loop(_BPW // _L // 4, _BPW // _L // 2, body, None)
        wb0.wait()
        pltpu.sync_copy(
            out_v.at[pl.ds(_HALF, _HALF)],
            out_hbm.at[b, pl.ds(blk0 * 1024 + _HALF, _HALF)],
        )


def kernel(x, table):
    xk = x.reshape(_BATCH, _NT, 128).transpose(1, 0, 2)
    tk = (
        jnp.transpose(table)
        .reshape(_D, _NT, 128)
        .transpose(1, 0, 2)
        .reshape(_TBL)
    )
    out_k = _gather_t(xk, tk)
    return (
        out_k.reshape(_BATCH, _NT, _D, 128)
        .transpose(0, 1, 3, 2)
        .reshape(_BATCH, _SEQ, _D)
    )


# speedup vs baseline: 1.0135x; 1.0135x over previous
import functools

import jax
import jax.numpy as jnp
from jax import lax
from jax.experimental import pallas as pl
from jax.experimental.pallas import tpu as pltpu
from jax.experimental.pallas import tpu_sc as plsc

_BATCH = 4
_SEQ = 8192
_D = 8
_NT = _SEQ // 128
_TBL = _D * _SEQ

_info = plsc.get_sparse_core_info()
_NC = _info.num_cores
_NS = _info.num_subcores
_NACT = 16
_BPW = _BATCH * _SEQ // _NACT
_WPR = _SEQ // _BPW
_BLK = _BPW // 128
_L = 16
_HALF = _BPW * _D // 2

_mesh = plsc.VectorSubcoreMesh(core_axis_name="c", subcore_axis_name="s")


@functools.partial(
    pl.kernel,
    mesh=_mesh,
    out_type=jax.ShapeDtypeStruct((_BATCH, _SEQ * _D), jnp.float32),
    scratch_types=[
        pltpu.VMEM((_BPW,), jnp.int32),
        pltpu.VMEM((_TBL,), jnp.float32),
        pltpu.VMEM((_BPW * _D,), jnp.float32),
        pltpu.SemaphoreType.DMA,
        pltpu.SemaphoreType.DMA,
    ],
    compiler_params=pltpu.CompilerParams(
        use_tc_tiling_on_sc=False, needs_layout_passes=False
    ),
)
def _gather_t(xk_hbm, tk_hbm, out_hbm, idx_v, tbl_v, out_v, sem, wsem):
    wid = lax.axis_index("s") * _NC + lax.axis_index("c")

    @pl.when(wid < _NACT)
    def _():
        b = wid // _WPR
        blk0 = (wid % _WPR) * _BLK
        cps = [pltpu.async_copy(tk_hbm, tbl_v, sem)]
        for i in range(_BLK):
            cps.append(
                pltpu.async_copy(
                    xk_hbm.at[blk0 + i, b], idx_v.at[pl.ds(i * 128, 128)], sem
                )
            )
        for cp in cps:
            cp.wait()

        def body(k, _):
            tvec = idx_v[pl.ds(pl.multiple_of(k * _L, _L), _L)]
            base = tvec + lax.shift_right_logical(tvec, 7) * 896
            obase = (k >> 3) * 1024 + (k & 7) * _L
            for d in range(_D):
                vals = plsc.load_gather(tbl_v, [base + d * 128])
                out_v[pl.ds(pl.multiple_of(obase + d * 128, _L), _L)] = vals
            return _

        lax.fori_loop(0, _BPW // _L // 2, body, None)
        wb0 = pltpu.async_copy(
            out_v.at[pl.ds(0, _HALF)],
            out_hbm.at[b, pl.ds(blk0 * 1024, _HALF)],
            wsem,
        )
        lax.fori_loop(_BPW // _L // 2, _BPW // _L, body, None)
        wb0.wait()
        pltpu.sync_copy(
            out_v.at[pl.ds(_HALF, _HALF)],
            out_hbm.at[b, pl.ds(blk0 * 1024 + _HALF, _HALF)],
        )


def kernel(x, table):
    xk = x.reshape(_BATCH, _NT, 128).transpose(1, 0, 2)
    tk = (
        jnp.transpose(table)
        .reshape(_D, _NT, 128)
        .transpose(1, 0, 2)
        .reshape(_TBL)
    )
    out_k = _gather_t(xk, tk)
    return (
        out_k.reshape(_BATCH, _NT, _D, 128)
        .transpose(0, 1, 3, 2)
        .reshape(_BATCH, _SEQ, _D)
    )
